# Initial kernel scaffold; baseline (speedup 1.0000x reference)
import jax
import jax.numpy as jnp
from jax.experimental import pallas as pl


def kernel(x, edge_index, Wl_head, bl_head, Wr_head, g_head, b_head, a_head, Wl_skip, bl_skip, Wr_skip, Wl_tail, bl_tail, Wr_tail):
    raise NotImplementedError("write your pallas kernel here")


# speedup vs baseline: 2.3638x; 2.3638x over previous
import functools

import jax
import jax.numpy as jnp
from jax import lax
from jax.experimental import pallas as pl
from jax.experimental.pallas import tpu as pltpu
from jax.experimental.pallas import tpu_sc as plsc

N = 10000
E = 320000
D = 128
NC = 2
NS = 16
W = NC * NS
RPW = 313
RB = RPW + 1
EB = 4000
NBLK = E // EB
FLUSH = 4096
LCAP = 158 * 2048
PACK = 16384
PADENT = RPW * PACK
NEG = -3.0e38
GC = 128
SCHUNK = 2048

_sc_mesh = plsc.VectorSubcoreMesh(
    core_axis_name="c", subcore_axis_name="s", num_cores=NC, num_subcores=NS)


def _wid():
    return lax.axis_index("s") * NC + lax.axis_index("c")


@functools.partial(
    pl.kernel,
    out_type=[jax.ShapeDtypeStruct((W, LCAP), jnp.int32),
              jax.ShapeDtypeStruct((W, 16), jnp.int32)],
    mesh=_sc_mesh,
    scratch_types=[pltpu.VMEM((EB,), jnp.int32),
                   pltpu.VMEM((EB,), jnp.int32),
                   pltpu.VMEM((2 * FLUSH,), jnp.int32),
                   pltpu.VMEM((16,), jnp.int32)],
    compiler_params=pltpu.CompilerParams(needs_layout_passes=False),
)
def _build_lists(src_hbm, dst_hbm, lists_hbm, counts_hbm, srcbuf, dstbuf,
                 packbuf, cbuf):
    w = _wid()
    lo = w * RPW

    def outer(b, carry):
        cnt, nflush = carry
        pltpu.sync_copy(src_hbm.at[pl.ds(b * EB, EB)], srcbuf)
        pltpu.sync_copy(dst_hbm.at[pl.ds(b * EB, EB)], dstbuf)

        def inner(j, cnt):
            dv = dstbuf[pl.ds(j * 16, 16)]
            sv = srcbuf[pl.ds(j * 16, 16)]
            m = (dv >= lo) & (dv < lo + RPW)
            mi = m.astype(jnp.int32)
            offs = cnt + plsc.cumsum(mi) - mi
            packed = (dv - lo) * PACK + sv
            plsc.store_scatter(packbuf, [offs], packed, mask=m)
            return cnt + jnp.sum(mi)

        cnt = lax.fori_loop(0, EB // 16, inner, cnt)
        flush = cnt >= FLUSH

        @pl.when(flush)
        def _():
            pltpu.sync_copy(packbuf.at[pl.ds(0, FLUSH)],
                            lists_hbm.at[w, pl.ds(nflush * FLUSH, FLUSH)])

            def mv(k, c):
                packbuf[pl.ds(k * 16, 16)] = packbuf[pl.ds(FLUSH + k * 16, 16)]
                return c

            lax.fori_loop(0, FLUSH // 16, mv, 0)

        cnt = jnp.where(flush, cnt - FLUSH, cnt)
        nflush = nflush + flush.astype(jnp.int32)
        return cnt, nflush

    cnt, nflush = lax.fori_loop(0, NBLK, outer,
                                (jnp.int32(0), jnp.int32(0)))
    padv = jnp.full((16,), PADENT, jnp.int32)
    for k in range(8):
        packbuf[pl.ds(cnt + k * 16, 16)] = padv
    pltpu.sync_copy(packbuf.at[pl.ds(0, FLUSH)],
                    lists_hbm.at[w, pl.ds(nflush * FLUSH, FLUSH)])
    padded = ((cnt + 127) // 128) * 128
    cbuf[...] = jnp.zeros((16,), jnp.int32) + (nflush * FLUSH + padded)
    pltpu.sync_copy(cbuf, counts_hbm.at[w])


@functools.partial(
    pl.kernel,
    out_type=jax.ShapeDtypeStruct((N * D,), jnp.float32),
    mesh=_sc_mesh,
    scratch_types=[pltpu.VMEM((RB * D,), jnp.float32),
                   pltpu.VMEM((SCHUNK,), jnp.int32),
                   pltpu.VMEM((GC,), jnp.int32),
                   pltpu.VMEM((GC, D), jnp.float32),
                   pltpu.VMEM((16,), jnp.int32),
                   pltpu.SemaphoreType.DMA],
    compiler_params=pltpu.CompilerParams(needs_layout_passes=False),
)
def _segmax(table_hbm, lists_hbm, counts_hbm, out_hbm,
            aggbuf, packchunk, srcidx, rows, cbuf, sem):
    w = _wid()
    negv = jnp.full((16,), NEG, jnp.float32)

    def init(i, c):
        aggbuf[pl.ds(i * 16, 16)] = negv
        return c

    lax.fori_loop(0, RB * D // 16, init, 0)

    pltpu.sync_copy(counts_hbm.at[w], cbuf)
    total = cbuf[...][0]
    nouter = (total + SCHUNK - 1) // SCHUNK

    def outer(g, c):
        pltpu.sync_copy(lists_hbm.at[w, pl.ds(g * SCHUNK, SCHUNK)], packchunk)

        def jloop(j, cc):
            @pl.when(g * SCHUNK + j * GC < total)
            def _():
                for k in range(GC // 16):
                    pv = packchunk[pl.ds(j * GC + k * 16, 16)]
                    srcidx[pl.ds(k * 16, 16)] = lax.rem(pv, PACK)
                pltpu.async_copy(table_hbm.at[srcidx], rows, sem).wait()

                def rloop(rr, ccc):
                    pv = packchunk[pl.ds(j * GC + rr * 16, 16)]
                    abv = lax.div(pv, PACK) * D
                    for lane in range(16):
                        ab = abv[lane]
                        r = rr * 16 + lane
                        for k in range(D // 16):
                            sl = pl.ds(ab + k * 16, 16)
                            aggbuf[sl] = jnp.maximum(
                                aggbuf[sl], rows[r, pl.ds(k * 16, 16)])
                    return ccc

                lax.fori_loop(0, GC // 16, rloop, 0)
            return cc

        lax.fori_loop(0, SCHUNK // GC, jloop, 0)
        return c

    lax.fori_loop(0, nouter, outer, 0)

    def fin(i, c):
        v = aggbuf[pl.ds(i * 16, 16)]
        aggbuf[pl.ds(i * 16, 16)] = jnp.where(v > -1.0e37, v, 0.0)
        return c

    lax.fori_loop(0, RPW * D // 16, fin, 0)

    @pl.when(w < W - 1)
    def _():
        pltpu.sync_copy(aggbuf.at[pl.ds(0, RPW * D)],
                        out_hbm.at[pl.ds(w * RPW * D, RPW * D)])

    @pl.when(w == W - 1)
    def _():
        nlast = (N - (W - 1) * RPW) * D
        pltpu.sync_copy(aggbuf.at[pl.ds(0, nlast)],
                        out_hbm.at[pl.ds((W - 1) * RPW * D, nlast)])


BR = 2000
GRID = N // BR


def _dotT(x, w):
    return lax.dot_general(x, w, (((1,), (1,)), ((), ())),
                           preferred_element_type=jnp.float32)


def _lin_body(agg_ref, h_ref, wl_ref, wr_ref, bl_ref, y_ref, s1_ref, s2_ref):
    i = pl.program_id(0)
    y = _dotT(agg_ref[...], wl_ref[...]) + _dotT(h_ref[...], wr_ref[...]) + bl_ref[...]
    y_ref[...] = y

    @pl.when(i == 0)
    def _():
        s1_ref[...] = jnp.zeros_like(s1_ref)
        s2_ref[...] = jnp.zeros_like(s2_ref)

    s1_ref[...] += jnp.sum(y, axis=0, keepdims=True)
    s2_ref[...] += jnp.sum(y * y, axis=0, keepdims=True)


def _lin(agg, h, Wl, bl, Wr):
    return pl.pallas_call(
        _lin_body,
        grid=(GRID,),
        in_specs=[pl.BlockSpec((BR, D), lambda i: (i, 0)),
                  pl.BlockSpec((BR, D), lambda i: (i, 0)),
                  pl.BlockSpec((D, D), lambda i: (0, 0)),
                  pl.BlockSpec((D, D), lambda i: (0, 0)),
                  pl.BlockSpec((1, D), lambda i: (0, 0))],
        out_specs=[pl.BlockSpec((BR, D), lambda i: (i, 0)),
                   pl.BlockSpec((1, D), lambda i: (0, 0)),
                   pl.BlockSpec((1, D), lambda i: (0, 0))],
        out_shape=[jax.ShapeDtypeStruct((N, D), jnp.float32),
                   jax.ShapeDtypeStruct((1, D), jnp.float32),
                   jax.ShapeDtypeStruct((1, D), jnp.float32)],
    )(agg, h, Wl, Wr, bl.reshape(1, D))


def _norm_body(y_ref, s1_ref, s2_ref, g_ref, b_ref, a_ref, o_ref):
    inv_n = 1.0 / N
    m = s1_ref[...] * inv_n
    ex2 = s2_ref[...] * inv_n
    am = a_ref[...] * m
    var = ex2 - 2.0 * am * m + am * am
    c = y_ref[...] - am
    o = g_ref[...] * c * lax.rsqrt(var + 1e-5) + b_ref[...]
    o_ref[...] = jnp.where(o >= 0, o, 0.02 * o)


def _norm(y, s1, s2, g, b, a):
    vec = pl.BlockSpec((1, D), lambda i: (0, 0))
    return pl.pallas_call(
        _norm_body,
        grid=(GRID,),
        in_specs=[pl.BlockSpec((BR, D), lambda i: (i, 0)),
                  vec, vec, vec, vec, vec],
        out_specs=pl.BlockSpec((BR, D), lambda i: (i, 0)),
        out_shape=jax.ShapeDtypeStruct((N, D), jnp.float32),
    )(y, s1, s2, g.reshape(1, D), b.reshape(1, D), a.reshape(1, D))


def _tail_body(ah_ref, as_ref, h_ref, s_ref, wl1, wl2, wr1, wr2, bl_ref, o_ref):
    t = (_dotT(ah_ref[...], wl1[...]) + _dotT(as_ref[...], wl2[...])
         + _dotT(h_ref[...], wr1[...]) + _dotT(s_ref[...], wr2[...])
         + bl_ref[...])
    o_ref[...] = jnp.tanh(t) * 0.5


def _tail(aggh, aggs, h, s, Wl_tail, bl_tail, Wr_tail):
    mat = pl.BlockSpec((BR, D), lambda i: (i, 0))
    wspec = pl.BlockSpec((D, D), lambda i: (0, 0))
    return pl.pallas_call(
        _tail_body,
        grid=(GRID,),
        in_specs=[mat, mat, mat, mat, wspec, wspec, wspec, wspec,
                  pl.BlockSpec((1, D), lambda i: (0, 0))],
        out_specs=mat,
        out_shape=jax.ShapeDtypeStruct((N, D), jnp.float32),
    )(aggh, aggs, h, s, Wl_tail[:, :D], Wl_tail[:, D:],
      Wr_tail[:, :D], Wr_tail[:, D:], bl_tail.reshape(1, D))


def kernel(x, edge_index, Wl_head, bl_head, Wr_head, g_head, b_head, a_head,
           Wl_skip, bl_skip, Wr_skip, Wl_tail, bl_tail, Wr_tail):
    lists, counts = _build_lists(edge_index[0], edge_index[1])

    h = x
    s = None
    for i in range(7):
        agg = _segmax(h, lists, counts).reshape(N, D)
        if i == 0:
            s, _, _ = _lin(agg, x, Wl_skip, bl_skip, Wr_skip)
        y, s1, s2 = _lin(agg, h, Wl_head[i], bl_head[i], Wr_head[i])
        if i < 6:
            h = _norm(y, s1, s2, g_head[i], b_head[i], a_head[i])
        else:
            h = y

    aggh = _segmax(h, lists, counts).reshape(N, D)
    aggs = _segmax(s, lists, counts).reshape(N, D)
    return _tail(aggh, aggs, h, s, Wl_tail, bl_tail, Wr_tail)
